# Initial kernel scaffold; baseline (speedup 1.0000x reference)
import jax
import jax.numpy as jnp
from jax.experimental import pallas as pl


def kernel(x, edge_index, conv0_W, conv0_b, conv1_W, conv1_b, conv2_W, conv2_b, att_W1, att_b1, att_W2, att_b2, cls_W1, cls_b1, cls_W2, cls_b2):
    raise NotImplementedError("write your pallas kernel here")


# speedup vs baseline: 12.6456x; 12.6456x over previous
import functools

import jax
import jax.numpy as jnp
from jax import lax
from jax.experimental import pallas as pl
from jax.experimental.pallas import tpu as pltpu
from jax.experimental.pallas import tpu_sc as plsc

N = 100000
E = 1600000
D_IN = 32
D_H = 64
NPAD = 100352
NC = 2
NS = 16
NR = NPAD // NS
CW = 16
NCHUNK = D_H // CW
CPS = NCHUNK // NC
EB = 1000
NB_AGG = (E // NS) // EB
EPC_DEG = E // NC
EPT_DEG = EPC_DEG // NS
NB_DEG = EPT_DEG // EB
BN = 6272
GRID = NPAD // BN


def _sc_mesh():
    return plsc.VectorSubcoreMesh(core_axis_name="c", subcore_axis_name="s")


def _sc_params():
    return pltpu.CompilerParams(use_tc_tiling_on_sc=False)


def _deg_call(col):
    ones = jnp.ones((EB,), jnp.float32)
    zeros = jnp.zeros((NR,), jnp.float32)

    @functools.partial(
        pl.kernel,
        out_type=jax.ShapeDtypeStruct((NC * NPAD,), jnp.float32),
        mesh=_sc_mesh(),
        scratch_types=[
            pltpu.VMEM((EB,), jnp.int32),
            pltpu.VMEM((EB,), jnp.float32),
            pltpu.VMEM_SHARED((NPAD,), jnp.float32),
        ],
        compiler_params=_sc_params(),
    )
    def deg_k(col_hbm, ones_hbm, zeros_hbm, out_hbm, idx_v, ones_v, acc_s):
        c = lax.axis_index("c")
        s = lax.axis_index("s")
        row0 = pl.multiple_of(s * NR, 128)
        pltpu.sync_copy(zeros_hbm, acc_s.at[pl.ds(row0, NR)])
        pltpu.sync_copy(ones_hbm, ones_v)
        plsc.subcore_barrier()
        ebase = c * EPC_DEG + s * EPT_DEG

        def body(i, carry):
            off = pl.multiple_of(ebase + i * EB, 8)
            pltpu.sync_copy(col_hbm.at[pl.ds(off, EB)], idx_v)
            pltpu.sync_copy(ones_v, acc_s.at[idx_v], add=True)
            return carry

        lax.fori_loop(0, NB_DEG, body, 0)
        plsc.subcore_barrier()
        out0 = pl.multiple_of(c * NPAD + row0, 128)
        pltpu.sync_copy(acc_s.at[pl.ds(row0, NR)], out_hbm.at[pl.ds(out0, NR)])

    return deg_k(col, ones, zeros).reshape(NC, NPAD)


def _agg_call(y, rid4, col):
    y16 = y.reshape(NCHUNK * NPAD, CW)
    zeros = jnp.zeros((NR, CW), jnp.float32)

    @functools.partial(
        pl.kernel,
        out_type=jax.ShapeDtypeStruct((NPAD, NCHUNK, CW), jnp.float32),
        mesh=_sc_mesh(),
        scratch_types=[
            pltpu.VMEM((EB,), jnp.int32),
            pltpu.VMEM((EB,), jnp.int32),
            pltpu.VMEM((EB, CW), jnp.float32),
            pltpu.VMEM_SHARED((NPAD, CW), jnp.float32),
            pltpu.SemaphoreType.DMA,
        ],
        compiler_params=_sc_params(),
    )
    def agg_k(y16_hbm, rid4_hbm, col_hbm, zeros_hbm, out_hbm,
              rid_v, cid_v, msg_v, acc_s, sem):
        c = lax.axis_index("c")
        s = lax.axis_index("s")
        row0 = pl.multiple_of(s * NR, 128)
        for sub in range(CPS):
            d = c * CPS + sub
            pltpu.sync_copy(zeros_hbm, acc_s.at[pl.ds(row0, NR)])
            plsc.subcore_barrier()
            ebase = s * (E // NS)

            def body(i, carry):
                off = pl.multiple_of(ebase + i * EB, 8)
                pltpu.sync_copy(rid4_hbm.at[d, pl.ds(off, EB)], rid_v)
                pltpu.sync_copy(col_hbm.at[pl.ds(off, EB)], cid_v)
                pltpu.async_copy(y16_hbm.at[rid_v], msg_v, sem).wait()
                pltpu.sync_copy(msg_v, acc_s.at[cid_v], add=True)
                return carry

            lax.fori_loop(0, NB_AGG, body, 0)
            plsc.subcore_barrier()
            pltpu.sync_copy(acc_s.at[pl.ds(row0, NR)],
                            out_hbm.at[pl.ds(row0, NR), d])
            plsc.subcore_barrier()

    return agg_k(y16, rid4, col, zeros).reshape(NPAD, D_H)


def _dinv_of(deg_ref):
    deg = deg_ref[0, :] + deg_ref[1, :] + 1.0
    return lax.rsqrt(jnp.maximum(deg, 1.0))


def _full(spec_shape):
    nd = len(spec_shape)
    return pl.BlockSpec(spec_shape, lambda i, _nd=nd: (0,) * _nd)


def _rows(width):
    return pl.BlockSpec((BN, width), lambda i: (i, 0))


_DEG_SPEC = pl.BlockSpec((NC, BN), lambda i: (0, i))


def _dense0_call(deg2, xtp, w0):
    def body(deg_ref, xt_ref, w_ref, y_ref):
        dinv = _dinv_of(deg_ref)
        xw = lax.dot_general(
            xt_ref[...], w_ref[...], (((0,), (0,)), ((), ())),
            preferred_element_type=jnp.float32)
        y_ref[...] = xw * dinv[:, None]

    return pl.pallas_call(
        body,
        grid=(GRID,),
        in_specs=[_DEG_SPEC,
                  pl.BlockSpec((D_IN, BN), lambda i: (0, i)),
                  _full((D_IN, D_H))],
        out_specs=_rows(D_H),
        out_shape=jax.ShapeDtypeStruct((NPAD, D_H), jnp.float32),
    )(deg2, xtp, w0)


def _dense_mid_call(deg2, agg, y, hprev, b, w_next):
    residual = hprev is not None

    def body(*refs):
        if residual:
            deg_ref, agg_ref, y_ref, hp_ref, b_ref, w_ref, h_ref, yn_ref = refs
        else:
            deg_ref, agg_ref, y_ref, b_ref, w_ref, h_ref, yn_ref = refs
        dinv = _dinv_of(deg_ref)
        h = jnp.maximum(
            (agg_ref[...] + y_ref[...]) * dinv[:, None] + b_ref[...], 0.0)
        if residual:
            h = hp_ref[...] + h
        h_ref[...] = h
        yn_ref[...] = (
            jnp.dot(h, w_ref[...], preferred_element_type=jnp.float32)
            * dinv[:, None])

    ins = [deg2, agg, y] + ([hprev] if residual else []) + [b, w_next]
    in_specs = (
        [_DEG_SPEC, _rows(D_H), _rows(D_H)]
        + ([_rows(D_H)] if residual else [])
        + [_full((1, D_H)), _full((D_H, D_H))]
    )
    return pl.pallas_call(
        body,
        grid=(GRID,),
        in_specs=in_specs,
        out_specs=[_rows(D_H), _rows(D_H)],
        out_shape=[
            jax.ShapeDtypeStruct((NPAD, D_H), jnp.float32),
            jax.ShapeDtypeStruct((NPAD, D_H), jnp.float32),
        ],
    )(*ins)


def _dense_final_call(deg2, agg, y, hprev, b2,
                      aw1, ab1, aw2, ab2, cw1, cb1, cw2, cb2):
    def body(deg_ref, agg_ref, y_ref, hp_ref, b_ref, aw1_ref, ab1_ref,
             aw2_ref, ab2_ref, cw1_ref, cb1_ref, cw2_ref, cb2_ref, out_ref):
        dinv = _dinv_of(deg_ref)
        h2 = hp_ref[...] + jnp.maximum(
            (agg_ref[...] + y_ref[...]) * dinv[:, None] + b_ref[...], 0.0)
        t = jnp.maximum(
            jnp.dot(h2, aw1_ref[...], preferred_element_type=jnp.float32)
            + ab1_ref[...], 0.0)
        u = jnp.dot(t, aw2_ref[...], preferred_element_type=jnp.float32) + ab2_ref[...]
        att = 1.0 / (1.0 + jnp.exp(-u))
        hf = h2 * att
        v = jnp.maximum(
            jnp.dot(hf, cw1_ref[...], preferred_element_type=jnp.float32)
            + cb1_ref[...], 0.0)
        logits = (
            jnp.dot(v, cw2_ref[...], preferred_element_type=jnp.float32)
            + cb2_ref[...])
        out_ref[...] = logits.T

    dh2 = D_H // 2
    return pl.pallas_call(
        body,
        grid=(GRID,),
        in_specs=[
            _DEG_SPEC, _rows(D_H), _rows(D_H), _rows(D_H), _full((1, D_H)),
            _full((D_H, dh2)), _full((1, dh2)), _full((dh2, 1)), _full((1, 1)),
            _full((D_H, dh2)), _full((1, dh2)), _full((dh2, 2)), _full((1, 2)),
        ],
        out_specs=pl.BlockSpec((2, BN), lambda i: (0, i)),
        out_shape=jax.ShapeDtypeStruct((2, NPAD), jnp.float32),
    )(deg2, agg, y, hprev, b2, aw1, ab1, aw2, ab2, cw1, cb1, cw2, cb2)


def kernel(x, edge_index, conv0_W, conv0_b, conv1_W, conv1_b, conv2_W, conv2_b,
           att_W1, att_b1, att_W2, att_b2, cls_W1, cls_b1, cls_W2, cls_b2):
    row = edge_index[0]
    col = edge_index[1]
    rid4 = 4 * row[None, :] + jnp.arange(NCHUNK, dtype=jnp.int32)[:, None]
    xtp = jnp.pad(x.T, ((0, 0), (0, NPAD - N)))

    deg2 = _deg_call(col)

    y0 = _dense0_call(deg2, xtp, conv0_W)
    agg0 = _agg_call(y0, rid4, col)
    h0, y1 = _dense_mid_call(deg2, agg0, y0, None, conv0_b.reshape(1, D_H),
                             conv1_W)
    agg1 = _agg_call(y1, rid4, col)
    h1, y2 = _dense_mid_call(deg2, agg1, y1, h0, conv1_b.reshape(1, D_H),
                             conv2_W)
    agg2 = _agg_call(y2, rid4, col)
    logits = _dense_final_call(
        deg2, agg2, y2, h1, conv2_b.reshape(1, D_H),
        att_W1, att_b1.reshape(1, D_H // 2), att_W2, att_b2.reshape(1, 1),
        cls_W1, cls_b1.reshape(1, D_H // 2), cls_W2, cls_b2.reshape(1, 2))
    return logits[:, :N].T


# speedup vs baseline: 16.3049x; 1.2894x over previous
import functools

import jax
import jax.numpy as jnp
from jax import lax
from jax.experimental import pallas as pl
from jax.experimental.pallas import tpu as pltpu
from jax.experimental.pallas import tpu_sc as plsc

N = 100000
E = 1600000
D_IN = 32
D_H = 64
NPAD = 100352
NC = 2
NS = 16
NR = NPAD // NS
CW = 16
NCHUNK = D_H // CW
CPS = NCHUNK // NC
EB = 800
NB_AGG = (E // NS) // EB
EBD = 1000
EPC_DEG = E // NC
EPT_DEG = EPC_DEG // NS
NB_DEG = EPT_DEG // EBD
BN = 6272
GRID = NPAD // BN


def _sc_mesh():
    return plsc.VectorSubcoreMesh(core_axis_name="c", subcore_axis_name="s")


def _sc_params():
    return pltpu.CompilerParams(use_tc_tiling_on_sc=False)


def _deg_call(col):
    ones = jnp.ones((EBD,), jnp.float32)
    zeros = jnp.zeros((NR,), jnp.float32)

    @functools.partial(
        pl.kernel,
        out_type=jax.ShapeDtypeStruct((NC * NPAD,), jnp.float32),
        mesh=_sc_mesh(),
        scratch_types=[
            pltpu.VMEM((EBD,), jnp.int32),
            pltpu.VMEM((EBD,), jnp.float32),
            pltpu.VMEM_SHARED((NPAD,), jnp.float32),
        ],
        compiler_params=_sc_params(),
    )
    def deg_k(col_hbm, ones_hbm, zeros_hbm, out_hbm, idx_v, ones_v, acc_s):
        c = lax.axis_index("c")
        s = lax.axis_index("s")
        row0 = pl.multiple_of(s * NR, 128)
        pltpu.sync_copy(zeros_hbm, acc_s.at[pl.ds(row0, NR)])
        pltpu.sync_copy(ones_hbm, ones_v)
        plsc.subcore_barrier()
        ebase = c * EPC_DEG + s * EPT_DEG

        def body(i, carry):
            off = pl.multiple_of(ebase + i * EBD, 8)
            pltpu.sync_copy(col_hbm.at[pl.ds(off, EBD)], idx_v)
            pltpu.sync_copy(ones_v, acc_s.at[idx_v], add=True)
            return carry

        lax.fori_loop(0, NB_DEG, body, 0)
        plsc.subcore_barrier()
        out0 = pl.multiple_of(c * NPAD + row0, 128)
        pltpu.sync_copy(acc_s.at[pl.ds(row0, NR)], out_hbm.at[pl.ds(out0, NR)])

    return deg_k(col, ones, zeros).reshape(NC, NPAD)


def _agg_call(y, rid4, col):
    y16 = y.reshape(NCHUNK * NPAD, CW)
    zeros = jnp.zeros((NR, CW), jnp.float32)

    @functools.partial(
        pl.kernel,
        out_type=jax.ShapeDtypeStruct((NPAD, NCHUNK, CW), jnp.float32),
        mesh=_sc_mesh(),
        scratch_types=[
            pltpu.VMEM((2, EB), jnp.int32),
            pltpu.VMEM((2, EB), jnp.int32),
            pltpu.VMEM((2, EB, CW), jnp.float32),
            pltpu.VMEM_SHARED((NPAD, CW), jnp.float32),
            pltpu.SemaphoreType.DMA,
            pltpu.SemaphoreType.DMA,
        ],
        compiler_params=_sc_params(),
    )
    def agg_k(y16_hbm, rid4_hbm, col_hbm, zeros_hbm, out_hbm,
              rid_v, cid_v, msg_v, acc_s, isem, gsem):
        c = lax.axis_index("c")
        s = lax.axis_index("s")
        row0 = pl.multiple_of(s * NR, 128)
        ebase = s * (E // NS)

        def idx_slices(i):
            off = pl.multiple_of(ebase + i * EB, 8)
            return rid4_hbm.at[d, pl.ds(off, EB)], col_hbm.at[pl.ds(off, EB)]

        for sub in range(CPS):
            d = c * CPS + sub
            pltpu.sync_copy(zeros_hbm, acc_s.at[pl.ds(row0, NR)])
            plsc.subcore_barrier()

            r0, c0 = idx_slices(0)
            pltpu.async_copy(r0, rid_v.at[0], isem)
            pltpu.async_copy(c0, cid_v.at[0], isem)

            def body(i, carry):
                b = lax.rem(i, 2)
                nb = 1 - b
                ri, ci = idx_slices(i)
                pltpu.make_async_copy(ri, rid_v.at[b], isem).wait()
                pltpu.make_async_copy(ci, cid_v.at[b], isem).wait()
                gd = pltpu.async_copy(
                    y16_hbm.at[rid_v.at[b]], msg_v.at[b], gsem)

                @pl.when(i > 0)
                def _():
                    pltpu.sync_copy(msg_v.at[nb], acc_s.at[cid_v.at[nb]],
                                    add=True)

                inext = jnp.minimum(i + 1, NB_AGG - 1)
                rn, cn = idx_slices(inext)
                pltpu.async_copy(rn, rid_v.at[nb], isem)
                pltpu.async_copy(cn, cid_v.at[nb], isem)
                gd.wait()
                return carry

            lax.fori_loop(0, NB_AGG, body, 0)
            lastb = (NB_AGG - 1) % 2
            rl, cl = idx_slices(NB_AGG - 1)
            pltpu.make_async_copy(rl, rid_v.at[1 - lastb], isem).wait()
            pltpu.make_async_copy(cl, cid_v.at[1 - lastb], isem).wait()
            pltpu.sync_copy(msg_v.at[lastb], acc_s.at[cid_v.at[lastb]],
                            add=True)
            plsc.subcore_barrier()
            pltpu.sync_copy(acc_s.at[pl.ds(row0, NR)],
                            out_hbm.at[pl.ds(row0, NR), d])
            plsc.subcore_barrier()

    return agg_k(y16, rid4, col, zeros).reshape(NPAD, D_H)


def _dinv_of(deg_ref):
    deg = deg_ref[0, :] + deg_ref[1, :] + 1.0
    return lax.rsqrt(jnp.maximum(deg, 1.0))


def _full(spec_shape):
    nd = len(spec_shape)
    return pl.BlockSpec(spec_shape, lambda i, _nd=nd: (0,) * _nd)


def _rows(width):
    return pl.BlockSpec((BN, width), lambda i: (i, 0))


_DEG_SPEC = pl.BlockSpec((NC, BN), lambda i: (0, i))


def _dense0_call(deg2, xtp, w0):
    def body(deg_ref, xt_ref, w_ref, y_ref):
        dinv = _dinv_of(deg_ref)
        xw = lax.dot_general(
            xt_ref[...], w_ref[...], (((0,), (0,)), ((), ())),
            preferred_element_type=jnp.float32)
        y_ref[...] = xw * dinv[:, None]

    return pl.pallas_call(
        body,
        grid=(GRID,),
        in_specs=[_DEG_SPEC,
                  pl.BlockSpec((D_IN, BN), lambda i: (0, i)),
                  _full((D_IN, D_H))],
        out_specs=_rows(D_H),
        out_shape=jax.ShapeDtypeStruct((NPAD, D_H), jnp.float32),
    )(deg2, xtp, w0)


def _dense_mid_call(deg2, agg, y, hprev, b, w_next):
    residual = hprev is not None

    def body(*refs):
        if residual:
            deg_ref, agg_ref, y_ref, hp_ref, b_ref, w_ref, h_ref, yn_ref = refs
        else:
            deg_ref, agg_ref, y_ref, b_ref, w_ref, h_ref, yn_ref = refs
        dinv = _dinv_of(deg_ref)
        h = jnp.maximum(
            (agg_ref[...] + y_ref[...]) * dinv[:, None] + b_ref[...], 0.0)
        if residual:
            h = hp_ref[...] + h
        h_ref[...] = h
        yn_ref[...] = (
            jnp.dot(h, w_ref[...], preferred_element_type=jnp.float32)
            * dinv[:, None])

    ins = [deg2, agg, y] + ([hprev] if residual else []) + [b, w_next]
    in_specs = (
        [_DEG_SPEC, _rows(D_H), _rows(D_H)]
        + ([_rows(D_H)] if residual else [])
        + [_full((1, D_H)), _full((D_H, D_H))]
    )
    return pl.pallas_call(
        body,
        grid=(GRID,),
        in_specs=in_specs,
        out_specs=[_rows(D_H), _rows(D_H)],
        out_shape=[
            jax.ShapeDtypeStruct((NPAD, D_H), jnp.float32),
            jax.ShapeDtypeStruct((NPAD, D_H), jnp.float32),
        ],
    )(*ins)


def _dense_final_call(deg2, agg, y, hprev, b2,
                      aw1, ab1, aw2, ab2, cw1, cb1, cw2, cb2):
    def body(deg_ref, agg_ref, y_ref, hp_ref, b_ref, aw1_ref, ab1_ref,
             aw2_ref, ab2_ref, cw1_ref, cb1_ref, cw2_ref, cb2_ref, out_ref):
        dinv = _dinv_of(deg_ref)
        h2 = hp_ref[...] + jnp.maximum(
            (agg_ref[...] + y_ref[...]) * dinv[:, None] + b_ref[...], 0.0)
        t = jnp.maximum(
            jnp.dot(h2, aw1_ref[...], preferred_element_type=jnp.float32)
            + ab1_ref[...], 0.0)
        u = jnp.dot(t, aw2_ref[...], preferred_element_type=jnp.float32) + ab2_ref[...]
        att = 1.0 / (1.0 + jnp.exp(-u))
        hf = h2 * att
        v = jnp.maximum(
            jnp.dot(hf, cw1_ref[...], preferred_element_type=jnp.float32)
            + cb1_ref[...], 0.0)
        logits = (
            jnp.dot(v, cw2_ref[...], preferred_element_type=jnp.float32)
            + cb2_ref[...])
        out_ref[...] = logits.T

    dh2 = D_H // 2
    return pl.pallas_call(
        body,
        grid=(GRID,),
        in_specs=[
            _DEG_SPEC, _rows(D_H), _rows(D_H), _rows(D_H), _full((1, D_H)),
            _full((D_H, dh2)), _full((1, dh2)), _full((dh2, 1)), _full((1, 1)),
            _full((D_H, dh2)), _full((1, dh2)), _full((dh2, 2)), _full((1, 2)),
        ],
        out_specs=pl.BlockSpec((2, BN), lambda i: (0, i)),
        out_shape=jax.ShapeDtypeStruct((2, NPAD), jnp.float32),
    )(deg2, agg, y, hprev, b2, aw1, ab1, aw2, ab2, cw1, cb1, cw2, cb2)


def kernel(x, edge_index, conv0_W, conv0_b, conv1_W, conv1_b, conv2_W, conv2_b,
           att_W1, att_b1, att_W2, att_b2, cls_W1, cls_b1, cls_W2, cls_b2):
    row = edge_index[0]
    col = edge_index[1]
    rid4 = 4 * row[None, :] + jnp.arange(NCHUNK, dtype=jnp.int32)[:, None]
    xtp = jnp.pad(x.T, ((0, 0), (0, NPAD - N)))

    deg2 = _deg_call(col)

    y0 = _dense0_call(deg2, xtp, conv0_W)
    agg0 = _agg_call(y0, rid4, col)
    h0, y1 = _dense_mid_call(deg2, agg0, y0, None, conv0_b.reshape(1, D_H),
                             conv1_W)
    agg1 = _agg_call(y1, rid4, col)
    h1, y2 = _dense_mid_call(deg2, agg1, y1, h0, conv1_b.reshape(1, D_H),
                             conv2_W)
    agg2 = _agg_call(y2, rid4, col)
    logits = _dense_final_call(
        deg2, agg2, y2, h1, conv2_b.reshape(1, D_H),
        att_W1, att_b1.reshape(1, D_H // 2), att_W2, att_b2.reshape(1, 1),
        cls_W1, cls_b1.reshape(1, D_H // 2), cls_W2, cls_b2.reshape(1, 2))
    return logits[:, :N].T


# speedup vs baseline: 19.8565x; 1.2178x over previous
import functools

import jax
import jax.numpy as jnp
from jax import lax
from jax.experimental import pallas as pl
from jax.experimental.pallas import tpu as pltpu
from jax.experimental.pallas import tpu_sc as plsc

N = 100000
E = 1600000
D_IN = 32
D_H = 64
NPAD = 100352
NC = 2
NS = 16
NR = NPAD // NS
CW = 16
NCHUNK = D_H // CW
CPS = NCHUNK // NC
EB = 800
NB_AGG = (E // NS) // EB
EBD = 1000
EPC_DEG = E // NC
EPT_DEG = EPC_DEG // NS
NB_DEG = EPT_DEG // EBD
BN = 6272
GRID = NPAD // BN


def _sc_mesh():
    return plsc.VectorSubcoreMesh(core_axis_name="c", subcore_axis_name="s")


def _sc_params():
    return pltpu.CompilerParams(use_tc_tiling_on_sc=False)


def _deg_call(col):
    ones = jnp.ones((EBD,), jnp.float32)
    zeros = jnp.zeros((NR,), jnp.float32)

    @functools.partial(
        pl.kernel,
        out_type=jax.ShapeDtypeStruct((NC * NPAD,), jnp.float32),
        mesh=_sc_mesh(),
        scratch_types=[
            pltpu.VMEM((EBD,), jnp.int32),
            pltpu.VMEM((EBD,), jnp.float32),
            pltpu.VMEM_SHARED((NPAD,), jnp.float32),
        ],
        compiler_params=_sc_params(),
    )
    def deg_k(col_hbm, ones_hbm, zeros_hbm, out_hbm, idx_v, ones_v, acc_s):
        c = lax.axis_index("c")
        s = lax.axis_index("s")
        row0 = pl.multiple_of(s * NR, 128)
        pltpu.sync_copy(zeros_hbm, acc_s.at[pl.ds(row0, NR)])
        pltpu.sync_copy(ones_hbm, ones_v)
        plsc.subcore_barrier()
        ebase = c * EPC_DEG + s * EPT_DEG

        def body(i, carry):
            off = pl.multiple_of(ebase + i * EBD, 8)
            pltpu.sync_copy(col_hbm.at[pl.ds(off, EBD)], idx_v)
            pltpu.sync_copy(ones_v, acc_s.at[idx_v], add=True)
            return carry

        lax.fori_loop(0, NB_DEG, body, 0)
        plsc.subcore_barrier()
        out0 = pl.multiple_of(c * NPAD + row0, 128)
        pltpu.sync_copy(acc_s.at[pl.ds(row0, NR)], out_hbm.at[pl.ds(out0, NR)])

    return deg_k(col, ones, zeros).reshape(NC, NPAD)


def _agg_call(y, rid4, col):
    y16 = y.reshape(NCHUNK * NPAD, CW)
    zeros = jnp.zeros((NR, CW), jnp.float32)

    @functools.partial(
        pl.kernel,
        out_type=jax.ShapeDtypeStruct((NPAD, D_H), jnp.float32),
        mesh=_sc_mesh(),
        scratch_types=[
            pltpu.VMEM((2, EB), jnp.int32),
            pltpu.VMEM((2, EB), jnp.int32),
            pltpu.VMEM((2, EB, CW), jnp.float32),
            pltpu.VMEM_SHARED((NPAD, CW), jnp.float32),
            pltpu.SemaphoreType.DMA,
            pltpu.SemaphoreType.DMA,
        ],
        compiler_params=_sc_params(),
    )
    def agg_k(y16_hbm, rid4_hbm, col_hbm, zeros_hbm, out_hbm,
              rid_v, cid_v, msg_v, acc_s, isem, gsem):
        c = lax.axis_index("c")
        s = lax.axis_index("s")
        row0 = pl.multiple_of(s * NR, 128)
        ebase = s * (E // NS)

        def idx_slices(i):
            off = pl.multiple_of(ebase + i * EB, 8)
            return rid4_hbm.at[d, pl.ds(off, EB)], col_hbm.at[pl.ds(off, EB)]

        for sub in range(CPS):
            d = c * CPS + sub
            pltpu.sync_copy(zeros_hbm, acc_s.at[pl.ds(row0, NR)])
            plsc.subcore_barrier()

            r0, c0 = idx_slices(0)
            pltpu.async_copy(r0, rid_v.at[0], isem)
            pltpu.async_copy(c0, cid_v.at[0], isem)

            def body(i, carry):
                b = lax.rem(i, 2)
                nb = 1 - b
                ri, ci = idx_slices(i)
                pltpu.make_async_copy(ri, rid_v.at[b], isem).wait()
                pltpu.make_async_copy(ci, cid_v.at[b], isem).wait()
                gd = pltpu.async_copy(
                    y16_hbm.at[rid_v.at[b]], msg_v.at[b], gsem)

                @pl.when(i > 0)
                def _():
                    pltpu.sync_copy(msg_v.at[nb], acc_s.at[cid_v.at[nb]],
                                    add=True)

                inext = jnp.minimum(i + 1, NB_AGG - 1)
                rn, cn = idx_slices(inext)
                pltpu.async_copy(rn, rid_v.at[nb], isem)
                pltpu.async_copy(cn, cid_v.at[nb], isem)
                gd.wait()
                return carry

            lax.fori_loop(0, NB_AGG, body, 0)
            lastb = (NB_AGG - 1) % 2
            rl, cl = idx_slices(NB_AGG - 1)
            pltpu.make_async_copy(rl, rid_v.at[1 - lastb], isem).wait()
            pltpu.make_async_copy(cl, cid_v.at[1 - lastb], isem).wait()
            pltpu.sync_copy(msg_v.at[lastb], acc_s.at[cid_v.at[lastb]],
                            add=True)
            plsc.subcore_barrier()
            pltpu.sync_copy(acc_s.at[pl.ds(row0, NR)],
                            out_hbm.at[pl.ds(row0, NR), pl.ds(CW * d, CW)])
            plsc.subcore_barrier()

    return agg_k(y16, rid4, col, zeros)


def _dinv_of(deg_ref):
    deg = deg_ref[0, :] + deg_ref[1, :] + 1.0
    return lax.rsqrt(jnp.maximum(deg, 1.0))


def _full(spec_shape):
    nd = len(spec_shape)
    return pl.BlockSpec(spec_shape, lambda i, _nd=nd: (0,) * _nd)


def _rows(width):
    return pl.BlockSpec((BN, width), lambda i: (i, 0))


_DEG_SPEC = pl.BlockSpec((NC, BN), lambda i: (0, i))


def _dense0_call(deg2, xtp, w0):
    def body(deg_ref, xt_ref, w_ref, y_ref):
        dinv = _dinv_of(deg_ref)
        xw = lax.dot_general(
            xt_ref[...], w_ref[...], (((0,), (0,)), ((), ())),
            preferred_element_type=jnp.float32)
        y_ref[...] = xw * dinv[:, None]

    return pl.pallas_call(
        body,
        grid=(GRID,),
        in_specs=[_DEG_SPEC,
                  pl.BlockSpec((D_IN, BN), lambda i: (0, i)),
                  _full((D_IN, D_H))],
        out_specs=_rows(D_H),
        out_shape=jax.ShapeDtypeStruct((NPAD, D_H), jnp.float32),
    )(deg2, xtp, w0)


def _dense_mid_call(deg2, agg, y, hprev, b, w_next):
    residual = hprev is not None

    def body(*refs):
        if residual:
            deg_ref, agg_ref, y_ref, hp_ref, b_ref, w_ref, h_ref, yn_ref = refs
        else:
            deg_ref, agg_ref, y_ref, b_ref, w_ref, h_ref, yn_ref = refs
        dinv = _dinv_of(deg_ref)
        h = jnp.maximum(
            (agg_ref[...] + y_ref[...]) * dinv[:, None] + b_ref[...], 0.0)
        if residual:
            h = hp_ref[...] + h
        h_ref[...] = h
        yn_ref[...] = (
            jnp.dot(h, w_ref[...], preferred_element_type=jnp.float32)
            * dinv[:, None])

    ins = [deg2, agg, y] + ([hprev] if residual else []) + [b, w_next]
    in_specs = (
        [_DEG_SPEC, _rows(D_H), _rows(D_H)]
        + ([_rows(D_H)] if residual else [])
        + [_full((1, D_H)), _full((D_H, D_H))]
    )
    return pl.pallas_call(
        body,
        grid=(GRID,),
        in_specs=in_specs,
        out_specs=[_rows(D_H), _rows(D_H)],
        out_shape=[
            jax.ShapeDtypeStruct((NPAD, D_H), jnp.float32),
            jax.ShapeDtypeStruct((NPAD, D_H), jnp.float32),
        ],
    )(*ins)


def _dense_final_call(deg2, agg, y, hprev, b2,
                      aw1, ab1, aw2, ab2, cw1, cb1, cw2, cb2):
    def body(deg_ref, agg_ref, y_ref, hp_ref, b_ref, aw1_ref, ab1_ref,
             aw2_ref, ab2_ref, cw1_ref, cb1_ref, cw2_ref, cb2_ref, out_ref):
        dinv = _dinv_of(deg_ref)
        h2 = hp_ref[...] + jnp.maximum(
            (agg_ref[...] + y_ref[...]) * dinv[:, None] + b_ref[...], 0.0)
        t = jnp.maximum(
            jnp.dot(h2, aw1_ref[...], preferred_element_type=jnp.float32)
            + ab1_ref[...], 0.0)
        u = jnp.dot(t, aw2_ref[...], preferred_element_type=jnp.float32) + ab2_ref[...]
        att = 1.0 / (1.0 + jnp.exp(-u))
        hf = h2 * att
        v = jnp.maximum(
            jnp.dot(hf, cw1_ref[...], preferred_element_type=jnp.float32)
            + cb1_ref[...], 0.0)
        logits = (
            jnp.dot(v, cw2_ref[...], preferred_element_type=jnp.float32)
            + cb2_ref[...])
        out_ref[...] = logits.T

    dh2 = D_H // 2
    return pl.pallas_call(
        body,
        grid=(GRID,),
        in_specs=[
            _DEG_SPEC, _rows(D_H), _rows(D_H), _rows(D_H), _full((1, D_H)),
            _full((D_H, dh2)), _full((1, dh2)), _full((dh2, 1)), _full((1, 1)),
            _full((D_H, dh2)), _full((1, dh2)), _full((dh2, 2)), _full((1, 2)),
        ],
        out_specs=pl.BlockSpec((2, BN), lambda i: (0, i)),
        out_shape=jax.ShapeDtypeStruct((2, NPAD), jnp.float32),
    )(deg2, agg, y, hprev, b2, aw1, ab1, aw2, ab2, cw1, cb1, cw2, cb2)


def kernel(x, edge_index, conv0_W, conv0_b, conv1_W, conv1_b, conv2_W, conv2_b,
           att_W1, att_b1, att_W2, att_b2, cls_W1, cls_b1, cls_W2, cls_b2):
    row = edge_index[0]
    col = edge_index[1]
    rid4 = 4 * row[None, :] + jnp.arange(NCHUNK, dtype=jnp.int32)[:, None]
    xtp = jnp.pad(x.T, ((0, 0), (0, NPAD - N)))

    deg2 = _deg_call(col)

    y0 = _dense0_call(deg2, xtp, conv0_W)
    agg0 = _agg_call(y0, rid4, col)
    h0, y1 = _dense_mid_call(deg2, agg0, y0, None, conv0_b.reshape(1, D_H),
                             conv1_W)
    agg1 = _agg_call(y1, rid4, col)
    h1, y2 = _dense_mid_call(deg2, agg1, y1, h0, conv1_b.reshape(1, D_H),
                             conv2_W)
    agg2 = _agg_call(y2, rid4, col)
    logits = _dense_final_call(
        deg2, agg2, y2, h1, conv2_b.reshape(1, D_H),
        att_W1, att_b1.reshape(1, D_H // 2), att_W2, att_b2.reshape(1, 1),
        cls_W1, cls_b1.reshape(1, D_H // 2), cls_W2, cls_b2.reshape(1, 2))
    return logits[:, :N].T


# speedup vs baseline: 23.7610x; 1.1966x over previous
import functools

import jax
import jax.numpy as jnp
from jax import lax
from jax.experimental import pallas as pl
from jax.experimental.pallas import tpu as pltpu
from jax.experimental.pallas import tpu_sc as plsc

N = 100000
E = 1600000
D_IN = 32
D_H = 64
NPAD = 100352
NC = 2
NS = 16
NR = NPAD // NS
CW = 16
NCHUNK = D_H // CW
CPS = NCHUNK // NC
EB = 800
NB_AGG = (E // NS) // EB
EBD = 1000
EPC_DEG = E // NC
EPT_DEG = EPC_DEG // NS
NB_DEG = EPT_DEG // EBD
BN = 6272
GRID = NPAD // BN


def _sc_mesh():
    return plsc.VectorSubcoreMesh(core_axis_name="c", subcore_axis_name="s")


def _sc_params():
    return pltpu.CompilerParams(use_tc_tiling_on_sc=False)


def _deg_call(col):
    ones = jnp.ones((EBD,), jnp.float32)
    zeros = jnp.zeros((NR,), jnp.float32)

    @functools.partial(
        pl.kernel,
        out_type=jax.ShapeDtypeStruct((NC * NPAD,), jnp.float32),
        mesh=_sc_mesh(),
        scratch_types=[
            pltpu.VMEM((EBD,), jnp.int32),
            pltpu.VMEM((EBD,), jnp.float32),
            pltpu.VMEM_SHARED((NPAD,), jnp.float32),
        ],
        compiler_params=_sc_params(),
    )
    def deg_k(col_hbm, ones_hbm, zeros_hbm, out_hbm, idx_v, ones_v, acc_s):
        c = lax.axis_index("c")
        s = lax.axis_index("s")
        row0 = pl.multiple_of(s * NR, 128)
        pltpu.sync_copy(zeros_hbm, acc_s.at[pl.ds(row0, NR)])
        pltpu.sync_copy(ones_hbm, ones_v)
        plsc.subcore_barrier()
        ebase = c * EPC_DEG + s * EPT_DEG

        def body(i, carry):
            off = pl.multiple_of(ebase + i * EBD, 8)
            pltpu.sync_copy(col_hbm.at[pl.ds(off, EBD)], idx_v)
            pltpu.sync_copy(ones_v, acc_s.at[idx_v], add=True)
            return carry

        lax.fori_loop(0, NB_DEG, body, 0)
        plsc.subcore_barrier()
        out0 = pl.multiple_of(c * NPAD + row0, 128)
        pltpu.sync_copy(acc_s.at[pl.ds(row0, NR)], out_hbm.at[pl.ds(out0, NR)])

    return deg_k(col, ones, zeros).reshape(NC, NPAD)


def _agg_call(y, rid4, col):
    y16 = y.reshape(NCHUNK * NPAD, CW)
    zeros = jnp.zeros((NR, CW), jnp.float32)

    @functools.partial(
        pl.kernel,
        out_type=jax.ShapeDtypeStruct((NPAD, D_H), jnp.float32),
        mesh=_sc_mesh(),
        scratch_types=[
            pltpu.VMEM((2, EB), jnp.int32),
            pltpu.VMEM((2, EB), jnp.int32),
            pltpu.VMEM((2, EB, CW), jnp.float32),
            pltpu.VMEM_SHARED((NPAD, CW), jnp.float32),
            pltpu.SemaphoreType.DMA,
            pltpu.SemaphoreType.DMA,
        ],
        compiler_params=_sc_params(),
    )
    def agg_k(y16_hbm, rid4_hbm, col_hbm, zeros_hbm, out_hbm,
              rid_v, cid_v, msg_v, acc_s, isem, gsem):
        c = lax.axis_index("c")
        s = lax.axis_index("s")
        row0 = pl.multiple_of(s * NR, 128)
        ebase = s * (E // NS)

        def idx_slices(i):
            off = pl.multiple_of(ebase + i * EB, 8)
            return rid4_hbm.at[d, pl.ds(off, EB)], col_hbm.at[pl.ds(off, EB)]

        for sub in range(CPS):
            d = c * CPS + sub
            pltpu.sync_copy(zeros_hbm, acc_s.at[pl.ds(row0, NR)])
            plsc.subcore_barrier()

            r0, c0 = idx_slices(0)
            pltpu.async_copy(r0, rid_v.at[0], isem)
            pltpu.async_copy(c0, cid_v.at[0], isem)

            def body(i, carry):
                b = lax.rem(i, 2)
                nb = 1 - b
                ri, ci = idx_slices(i)
                pltpu.make_async_copy(ri, rid_v.at[b], isem).wait()
                pltpu.make_async_copy(ci, cid_v.at[b], isem).wait()
                gd = pltpu.async_copy(
                    y16_hbm.at[rid_v.at[b]], msg_v.at[b], gsem)

                @pl.when(i > 0)
                def _():
                    pltpu.sync_copy(msg_v.at[nb], acc_s.at[cid_v.at[nb]],
                                    add=True)

                inext = jnp.minimum(i + 1, NB_AGG - 1)
                rn, cn = idx_slices(inext)
                pltpu.async_copy(rn, rid_v.at[nb], isem)
                pltpu.async_copy(cn, cid_v.at[nb], isem)
                gd.wait()
                return carry

            lax.fori_loop(0, NB_AGG, body, 0)
            lastb = (NB_AGG - 1) % 2
            rl, cl = idx_slices(NB_AGG - 1)
            pltpu.make_async_copy(rl, rid_v.at[1 - lastb], isem).wait()
            pltpu.make_async_copy(cl, cid_v.at[1 - lastb], isem).wait()
            pltpu.sync_copy(msg_v.at[lastb], acc_s.at[cid_v.at[lastb]],
                            add=True)
            plsc.subcore_barrier()
            pltpu.sync_copy(acc_s.at[pl.ds(row0, NR)],
                            out_hbm.at[pl.ds(row0, NR), pl.ds(CW * d, CW)])
            plsc.subcore_barrier()

    return agg_k(y16, rid4, col, zeros)


EPAD = 1638400


def _rid4_call(row):
    be = 16384
    rowp = jnp.pad(row, (0, EPAD - E))

    def body(row_ref, out_ref):
        r = row_ref[...]
        out_ref[...] = 4 * r[None, :] + lax.broadcasted_iota(
            jnp.int32, (NCHUNK, be), 0)

    return pl.pallas_call(
        body,
        grid=(EPAD // be,),
        in_specs=[pl.BlockSpec((be,), lambda i: (i,))],
        out_specs=pl.BlockSpec((NCHUNK, be), lambda i: (0, i)),
        out_shape=jax.ShapeDtypeStruct((NCHUNK, EPAD), jnp.int32),
    )(rowp)


def _dinv_of(deg_ref):
    deg = deg_ref[0, :] + deg_ref[1, :] + 1.0
    return lax.rsqrt(jnp.maximum(deg, 1.0))


def _full(spec_shape):
    nd = len(spec_shape)
    return pl.BlockSpec(spec_shape, lambda i, _nd=nd: (0,) * _nd)


def _rows(width):
    return pl.BlockSpec((BN, width), lambda i: (i, 0))


_DEG_SPEC = pl.BlockSpec((NC, BN), lambda i: (0, i))


def _dense0_call(deg2, xtp, w0):
    def body(deg_ref, xt_ref, w_ref, y_ref):
        dinv = _dinv_of(deg_ref)
        xw = lax.dot_general(
            xt_ref[...], w_ref[...], (((0,), (0,)), ((), ())),
            preferred_element_type=jnp.float32)
        y_ref[...] = xw * dinv[:, None]

    return pl.pallas_call(
        body,
        grid=(GRID,),
        in_specs=[_DEG_SPEC,
                  pl.BlockSpec((D_IN, BN), lambda i: (0, i)),
                  _full((D_IN, D_H))],
        out_specs=_rows(D_H),
        out_shape=jax.ShapeDtypeStruct((NPAD, D_H), jnp.float32),
    )(deg2, xtp, w0)


def _dense_mid_call(deg2, agg, y, hprev, b, w_next):
    residual = hprev is not None

    def body(*refs):
        if residual:
            deg_ref, agg_ref, y_ref, hp_ref, b_ref, w_ref, h_ref, yn_ref = refs
        else:
            deg_ref, agg_ref, y_ref, b_ref, w_ref, h_ref, yn_ref = refs
        dinv = _dinv_of(deg_ref)
        h = jnp.maximum(
            (agg_ref[...] + y_ref[...]) * dinv[:, None] + b_ref[...], 0.0)
        if residual:
            h = hp_ref[...] + h
        h_ref[...] = h
        yn_ref[...] = (
            jnp.dot(h, w_ref[...], preferred_element_type=jnp.float32)
            * dinv[:, None])

    ins = [deg2, agg, y] + ([hprev] if residual else []) + [b, w_next]
    in_specs = (
        [_DEG_SPEC, _rows(D_H), _rows(D_H)]
        + ([_rows(D_H)] if residual else [])
        + [_full((1, D_H)), _full((D_H, D_H))]
    )
    return pl.pallas_call(
        body,
        grid=(GRID,),
        in_specs=in_specs,
        out_specs=[_rows(D_H), _rows(D_H)],
        out_shape=[
            jax.ShapeDtypeStruct((NPAD, D_H), jnp.float32),
            jax.ShapeDtypeStruct((NPAD, D_H), jnp.float32),
        ],
    )(*ins)


def _dense_final_call(deg2, agg, y, hprev, b2,
                      aw1, ab1, aw2, ab2, cw1, cb1, cw2, cb2):
    def body(deg_ref, agg_ref, y_ref, hp_ref, b_ref, aw1_ref, ab1_ref,
             aw2_ref, ab2_ref, cw1_ref, cb1_ref, cw2_ref, cb2_ref, out_ref):
        dinv = _dinv_of(deg_ref)
        h2 = hp_ref[...] + jnp.maximum(
            (agg_ref[...] + y_ref[...]) * dinv[:, None] + b_ref[...], 0.0)
        t = jnp.maximum(
            jnp.dot(h2, aw1_ref[...], preferred_element_type=jnp.float32)
            + ab1_ref[...], 0.0)
        u = jnp.dot(t, aw2_ref[...], preferred_element_type=jnp.float32) + ab2_ref[...]
        att = 1.0 / (1.0 + jnp.exp(-u))
        hf = h2 * att
        v = jnp.maximum(
            jnp.dot(hf, cw1_ref[...], preferred_element_type=jnp.float32)
            + cb1_ref[...], 0.0)
        logits = (
            jnp.dot(v, cw2_ref[...], preferred_element_type=jnp.float32)
            + cb2_ref[...])
        out_ref[...] = logits.T

    dh2 = D_H // 2
    return pl.pallas_call(
        body,
        grid=(GRID,),
        in_specs=[
            _DEG_SPEC, _rows(D_H), _rows(D_H), _rows(D_H), _full((1, D_H)),
            _full((D_H, dh2)), _full((1, dh2)), _full((dh2, 1)), _full((1, 1)),
            _full((D_H, dh2)), _full((1, dh2)), _full((dh2, 2)), _full((1, 2)),
        ],
        out_specs=pl.BlockSpec((2, BN), lambda i: (0, i)),
        out_shape=jax.ShapeDtypeStruct((2, NPAD), jnp.float32),
    )(deg2, agg, y, hprev, b2, aw1, ab1, aw2, ab2, cw1, cb1, cw2, cb2)


def kernel(x, edge_index, conv0_W, conv0_b, conv1_W, conv1_b, conv2_W, conv2_b,
           att_W1, att_b1, att_W2, att_b2, cls_W1, cls_b1, cls_W2, cls_b2):
    row = edge_index[0]
    col = edge_index[1]
    rid4 = _rid4_call(row)
    xtp = jnp.pad(x.T, ((0, 0), (0, NPAD - N)))

    deg2 = _deg_call(col)

    y0 = _dense0_call(deg2, xtp, conv0_W)
    agg0 = _agg_call(y0, rid4, col)
    h0, y1 = _dense_mid_call(deg2, agg0, y0, None, conv0_b.reshape(1, D_H),
                             conv1_W)
    agg1 = _agg_call(y1, rid4, col)
    h1, y2 = _dense_mid_call(deg2, agg1, y1, h0, conv1_b.reshape(1, D_H),
                             conv2_W)
    agg2 = _agg_call(y2, rid4, col)
    logits = _dense_final_call(
        deg2, agg2, y2, h1, conv2_b.reshape(1, D_H),
        att_W1, att_b1.reshape(1, D_H // 2), att_W2, att_b2.reshape(1, 1),
        cls_W1, cls_b1.reshape(1, D_H // 2), cls_W2, cls_b2.reshape(1, 2))
    return logits[:, :N].T
